# baseline (speedup 1.0000x reference)
import jax
import jax.numpy as jnp
from jax import lax
from jax.experimental import pallas as pl
from jax.experimental.pallas import tpu as pltpu
from jax.experimental.pallas import tpu_sc as plsc

_N = 10000
_E = 320000
_D = 128
_H = 64
_G = 256

_NC = 2
_NS = 16
_NW = _NC * _NS
_EPW = _E // _NW
_RPT = _N // _NS

_BLK = 1000
_NBLK = _N // _BLK


def _make_seg_sum(width, chunk, k):
    nchunk = _EPW // chunk
    ngrp = nchunk // k

    def body(feat, src2, dst2, zinit, out, src_v, dst_v, rows, acc,
             gsem, ssem):
        c = lax.axis_index("c")
        s = lax.axis_index("s")
        wid = c * _NS + s

        pltpu.sync_copy(zinit, acc.at[pl.ds(s * _RPT, _RPT)])
        pltpu.sync_copy(src2.at[wid], src_v)
        pltpu.sync_copy(dst2.at[wid], dst_v)
        plsc.subcore_barrier()

        def gather_wait(slot):
            pltpu.make_async_copy(feat.at[src_v.at[0]], rows.at[slot],
                                  gsem.at[slot]).wait()

        def scatter_wait(slot):
            pltpu.make_async_copy(rows.at[slot], acc.at[dst_v.at[0]],
                                  ssem.at[slot]).wait()

        for b in range(k):
            pltpu.async_copy(feat.at[src_v.at[b]], rows.at[b], gsem.at[b])

        def outer(j, carry):
            for h in (0, 1):
                g = 2 * j + h

                @pl.when(g < ngrp)
                def _():
                    for b in range(k):
                        slot = h * k + b
                        gather_wait(slot)
                        pltpu.async_copy(rows.at[slot],
                                         acc.at[dst_v.at[g * k + b]],
                                         ssem.at[slot], add=True)

                    @pl.when(g + 1 < ngrp)
                    def _():
                        for b in range(k):
                            slot = (1 - h) * k + b

                            @pl.when(g >= 1)
                            def _():
                                scatter_wait(slot)

                            pltpu.async_copy(
                                feat.at[src_v.at[(g + 1) * k + b]],
                                rows.at[slot], gsem.at[slot])
            return carry

        lax.fori_loop(0, (ngrp + 2) // 2, outer, 0)
        for b in range(2 * k):
            scatter_wait(b)
        plsc.subcore_barrier()
        pltpu.sync_copy(acc.at[pl.ds(s * _RPT, _RPT)],
                        out.at[c].at[pl.ds(s * _RPT, _RPT)])

    return pl.kernel(
        body,
        out_type=jax.ShapeDtypeStruct((_NC, _N, width), jnp.float32),
        mesh=plsc.VectorSubcoreMesh(core_axis_name="c",
                                    subcore_axis_name="s"),
        scratch_types=[
            pltpu.VMEM((nchunk, chunk), jnp.int32),
            pltpu.VMEM((nchunk, chunk), jnp.int32),
            pltpu.VMEM((2 * k, chunk, width), jnp.float32),
            pltpu.VMEM_SHARED((_N, width), jnp.float32),
            pltpu.SemaphoreType.DMA((2 * k,)),
            pltpu.SemaphoreType.DMA((2 * k,)),
        ],
        compiler_params=pltpu.CompilerParams(use_tc_tiling_on_sc=False),
    )


_seg_sum_128 = _make_seg_sum(_D, 40, 2)
_seg_sum_64 = _make_seg_sum(_H, 80, 5)


def _mid_body(x_ref, p_ref, w1a_ref, b1a_ref, w1b_ref, b1b_ref, o_ref):
    z = jax.lax.dot_general(x_ref[...] + (p_ref[0] + p_ref[1]), w1a_ref[...],
                            (((1,), (0,)), ((), ())),
                            preferred_element_type=jnp.float32)
    z = jnp.maximum(z + b1a_ref[...], 0.0)
    h = jax.lax.dot_general(z, w1b_ref[...], (((1,), (0,)), ((), ())),
                            preferred_element_type=jnp.float32)
    o_ref[...] = jnp.maximum(h + b1b_ref[...], 0.0)


def _mid(x, parts, w1a, b1a, w1b, b1b):
    return pl.pallas_call(
        _mid_body,
        grid=(_NBLK,),
        in_specs=[
            pl.BlockSpec((_BLK, _D), lambda i: (i, 0)),
            pl.BlockSpec((_NC, _BLK, _D), lambda i: (0, i, 0)),
            pl.BlockSpec((_D, _H), lambda i: (0, 0)),
            pl.BlockSpec((1, _H), lambda i: (0, 0)),
            pl.BlockSpec((_H, _H), lambda i: (0, 0)),
            pl.BlockSpec((1, _H), lambda i: (0, 0)),
        ],
        out_specs=pl.BlockSpec((_BLK, _H), lambda i: (i, 0)),
        out_shape=jax.ShapeDtypeStruct((_N, _H), jnp.float32),
    )(x, parts, w1a, b1a, w1b, b1b)


def _pool_body(h1_ref, q_ref, w2a_ref, b2a_ref, w2b_ref, b2b_ref, batch_ref,
               we_ref, be_ref, wp_ref, bp_ref,
               hg_ref, e_ref, p_ref, acc_ref, cnt_ref):
    i = pl.program_id(0)

    z = jax.lax.dot_general(h1_ref[...] + (q_ref[0] + q_ref[1]), w2a_ref[...],
                            (((1,), (0,)), ((), ())),
                            preferred_element_type=jnp.float32)
    z = jnp.maximum(z + b2a_ref[...], 0.0)
    h2 = jax.lax.dot_general(z, w2b_ref[...], (((1,), (0,)), ((), ())),
                             preferred_element_type=jnp.float32)
    h2 = jnp.maximum(h2 + b2b_ref[...], 0.0)

    gids = jax.lax.broadcasted_iota(jnp.int32, (1, _G), 1)
    m = (batch_ref[...] == gids).astype(jnp.float32)

    @pl.when(i == 0)
    def _():
        acc_ref[...] = jnp.zeros_like(acc_ref)
        cnt_ref[...] = jnp.zeros_like(cnt_ref)

    acc_ref[...] += jax.lax.dot_general(
        m, h2, (((0,), (0,)), ((), ())), preferred_element_type=jnp.float32,
        precision=jax.lax.Precision.HIGHEST)
    cnt_ref[...] += jax.lax.dot_general(
        m, jnp.ones((_BLK, 1), jnp.float32), (((0,), (0,)), ((), ())),
        preferred_element_type=jnp.float32,
        precision=jax.lax.Precision.HIGHEST)

    @pl.when(i == _NBLK - 1)
    def _():
        hg = acc_ref[...] / jnp.maximum(cnt_ref[...], 1.0)
        hg_ref[...] = hg
        e_ref[...] = jax.lax.dot_general(
            hg, we_ref[...], (((1,), (0,)), ((), ())),
            preferred_element_type=jnp.float32) + be_ref[...]
        p_ref[...] = jax.lax.dot_general(
            hg, wp_ref[...], (((1,), (0,)), ((), ())),
            preferred_element_type=jnp.float32) + bp_ref[...]


def _pool(h1, parts, w2a, b2a, w2b, b2b, batch2, we, be, wp, bp):
    return pl.pallas_call(
        _pool_body,
        grid=(_NBLK,),
        in_specs=[
            pl.BlockSpec((_BLK, _H), lambda i: (i, 0)),
            pl.BlockSpec((_NC, _BLK, _H), lambda i: (0, i, 0)),
            pl.BlockSpec((_H, _H), lambda i: (0, 0)),
            pl.BlockSpec((1, _H), lambda i: (0, 0)),
            pl.BlockSpec((_H, _H), lambda i: (0, 0)),
            pl.BlockSpec((1, _H), lambda i: (0, 0)),
            pl.BlockSpec((_BLK, 1), lambda i: (i, 0)),
            pl.BlockSpec((_H, 1), lambda i: (0, 0)),
            pl.BlockSpec((1, 1), lambda i: (0, 0)),
            pl.BlockSpec((_H, 6), lambda i: (0, 0)),
            pl.BlockSpec((1, 6), lambda i: (0, 0)),
        ],
        out_specs=[
            pl.BlockSpec((_G, _H), lambda i: (0, 0)),
            pl.BlockSpec((_G, 1), lambda i: (0, 0)),
            pl.BlockSpec((_G, 6), lambda i: (0, 0)),
        ],
        out_shape=[
            jax.ShapeDtypeStruct((_G, _H), jnp.float32),
            jax.ShapeDtypeStruct((_G, 1), jnp.float32),
            jax.ShapeDtypeStruct((_G, 6), jnp.float32),
        ],
        scratch_shapes=[
            pltpu.VMEM((_G, _H), jnp.float32),
            pltpu.VMEM((_G, 1), jnp.float32),
        ],
    )(h1, parts, w2a, b2a, w2b, b2b, batch2, we, be, wp, bp)


@jax.jit
def kernel(x, edge_index, batch, W1a, b1a, W1b, b1b, W2a, b2a, W2b, b2b,
           We, be, Wp, bp):
    src40 = edge_index[0].reshape(_NW, _EPW // 40, 40)
    dst40 = edge_index[1].reshape(_NW, _EPW // 40, 40)
    src80 = edge_index[0].reshape(_NW, _EPW // 80, 80)
    dst80 = edge_index[1].reshape(_NW, _EPW // 80, 80)
    z128 = jnp.zeros((_RPT, _D), jnp.float32)
    z64 = jnp.zeros((_RPT, _H), jnp.float32)
    batch2 = batch.reshape(_N, 1)

    p1 = _seg_sum_128(x, src40, dst40, z128)
    h1 = _mid(x, p1, W1a, b1a.reshape(1, _H), W1b, b1b.reshape(1, _H))
    p2 = _seg_sum_64(h1, src80, dst80, z64)
    hg, e, p = _pool(h1, p2, W2a, b2a.reshape(1, _H), W2b,
                     b2b.reshape(1, _H), batch2, We, be.reshape(1, 1), Wp,
                     bp.reshape(1, 6))
    return hg, e, p
